# baseline (speedup 1.0000x reference)
import functools

import jax
import jax.numpy as jnp
from jax import lax
from jax.experimental import pallas as pl
from jax.experimental.pallas import tpu as pltpu
from jax.experimental.pallas import tpu_sc as plsc

NUM_FEATS = 26
DIM = 32
CARD = 100000
BATCH = 16384
NC = 2
NS = 16
NW = NC * NS
BPW = BATCH // NW

LANES_MAIN = 780 * 128
BIGT = 13
BIGL = BIGT * 128
NBIG = LANES_MAIN // BIGL
REMT_L0 = LANES_MAIN
QROWS = 25024


def _detile(*args):
    mesh = plsc.VectorSubcoreMesh(core_axis_name="c", subcore_axis_name="s")

    @functools.partial(
        pl.kernel,
        mesh=mesh,
        out_type=tuple(
            jax.ShapeDtypeStruct((QROWS, 4 * DIM), jnp.float32)
            for _ in range(NUM_FEATS)
        ),
        scratch_types=[
            pltpu.VMEM((DIM, BIGL), jnp.float32),
            pltpu.VMEM((BIGL // 4, 4 * DIM), jnp.float32),
        ],
        compiler_params=pltpu.CompilerParams(
            use_tc_tiling_on_sc=True, needs_layout_passes=False
        ),
    )
    def ka(*refs):
        wt_refs = refs[:NUM_FEATS]
        tail_ref = refs[NUM_FEATS]
        wlin_refs = refs[NUM_FEATS + 1:2 * NUM_FEATS + 1]
        slab_v, out_v = refs[2 * NUM_FEATS + 1:]

        wid = lax.axis_index("s") * NC + lax.axis_index("c")

        def transpose_groups(src_v, dst_v, ngroups):
            @pl.loop(0, ngroups)
            def _(g):
                lane = jax.lax.iota(jnp.int32, 16) + g * 16
                rowid = lax.shift_right_logical(lane, 2)
                colbase = lax.shift_left(lane & 3, 5)

                @pl.loop(0, 2)
                def _(ch):
                    for i in range(DIM // 2):
                        c = ch * (DIM // 2) + i
                        v = src_v[c, pl.ds(g * 16, 16)]
                        plsc.store_scatter(dst_v, [rowid, colbase + c], v)

        for f in range(NUM_FEATS):
            @pl.loop(0, 2)
            def _(rep, f=f):
                cidx = wid + rep * NW

                @pl.when(cidx < NBIG)
                def _():
                    pltpu.sync_copy(
                        wt_refs[f].at[:, pl.ds(cidx * BIGL, BIGL)], slab_v
                    )
                    transpose_groups(slab_v, out_v, BIGL // 16)
                    pltpu.sync_copy(
                        out_v,
                        wlin_refs[f].at[
                            pl.ds(cidx * (BIGL // 4), BIGL // 4), :
                        ],
                    )

            w1 = f % NW
            w2 = (f + 13) % NW

            @pl.when((wid == w1) | (wid == w2))
            def _(f=f, w1=w1):
                @pl.when(wid == w1)
                def _():
                    pltpu.sync_copy(
                        wt_refs[f].at[:, pl.ds(REMT_L0, 128)],
                        slab_v.at[:, pl.ds(0, 128)],
                    )

                @pl.when(wid != w1)
                def _():
                    pltpu.sync_copy(tail_ref.at[f], slab_v.at[:, pl.ds(0, 128)])

                transpose_groups(slab_v, out_v, 128 // 16)
                q0 = jnp.where(wid == w1, REMT_L0 // 4, 24992)
                q0 = pl.multiple_of(q0, 8)
                pltpu.sync_copy(
                    out_v.at[pl.ds(0, 32), :],
                    wlin_refs[f].at[pl.ds(q0, 32), :],
                )

    return ka(*args)


def _gather(*args):
    mesh = plsc.VectorSubcoreMesh(core_axis_name="c", subcore_axis_name="s")

    @functools.partial(
        pl.kernel,
        mesh=mesh,
        out_type=jax.ShapeDtypeStruct((BATCH, NUM_FEATS * DIM), jnp.float32),
        scratch_types=[
            pltpu.VMEM((BPW,), jnp.int32),
            pltpu.VMEM((BPW, DIM), jnp.float32),
            pltpu.SemaphoreType.DMA,
        ],
        compiler_params=pltpu.CompilerParams(use_tc_tiling_on_sc=False),
    )
    def kb(*refs):
        idx_refs = refs[:NUM_FEATS]
        tbl_refs = refs[NUM_FEATS:2 * NUM_FEATS]
        out = refs[2 * NUM_FEATS]
        idx_v, rows_v, sem = refs[2 * NUM_FEATS + 1:]

        wid = lax.axis_index("s") * NC + lax.axis_index("c")
        base = wid * BPW
        for f in range(NUM_FEATS):
            pltpu.sync_copy(idx_refs[f].at[pl.ds(base, BPW)], idx_v)
            pltpu.async_copy(tbl_refs[f].at[idx_v], rows_v, sem).wait()
            pltpu.sync_copy(
                rows_v, out.at[pl.ds(base, BPW), pl.ds(f * DIM, DIM)]
            )

    return kb(*args)


def _cat_embed(*args):
    idx_args = args[:NUM_FEATS]
    w_args = args[NUM_FEATS:]
    wt_args = [jnp.swapaxes(w, 0, 1) for w in w_args]
    tail = jnp.pad(
        jnp.stack([w[781 * 128:, :].T for w in w_args]),
        ((0, 0), (0, 0), (0, 128 - DIM)),
    )
    wlins = _detile(*wt_args, tail)
    tbls = [wl.reshape(4 * QROWS, DIM) for wl in wlins]
    return _gather(*idx_args, *tbls)


def kernel(f00, f01, f02, f03, f04, f05, f06, f07, f08, f09, f10, f11, f12,
           f13, f14, f15, f16, f17, f18, f19, f20, f21, f22, f23, f24, f25,
           W_f00, W_f01, W_f02, W_f03, W_f04, W_f05, W_f06, W_f07, W_f08,
           W_f09, W_f10, W_f11, W_f12, W_f13, W_f14, W_f15, W_f16, W_f17,
           W_f18, W_f19, W_f20, W_f21, W_f22, W_f23, W_f24, W_f25):
    return _cat_embed(
        f00, f01, f02, f03, f04, f05, f06, f07, f08, f09, f10, f11, f12,
        f13, f14, f15, f16, f17, f18, f19, f20, f21, f22, f23, f24, f25,
        W_f00, W_f01, W_f02, W_f03, W_f04, W_f05, W_f06, W_f07, W_f08,
        W_f09, W_f10, W_f11, W_f12, W_f13, W_f14, W_f15, W_f16, W_f17,
        W_f18, W_f19, W_f20, W_f21, W_f22, W_f23, W_f24, W_f25)
